# baseline (speedup 1.0000x reference)
import functools

import jax
import jax.numpy as jnp
from jax import lax
from jax.experimental import pallas as pl
from jax.experimental.pallas import tpu as pltpu
from jax.experimental.pallas import tpu_sc as plsc

NUM_VOCAB = 1000000
D = 64
BATCH = 4096
SEQ = 200

NC = 2
NS = 16
NW = NC * NS

B_TOTAL = BATCH * SEQ
ROWS_PER_W = B_TOTAL // NW
CH = SEQ
NSTEP = ROWS_PER_W // CH
NBUF = 4


def _emb_body(x_hbm, emb_hbm, pos_hbm, out_hbm,
              idx_v, pos_v, rows_v, gsem, osem):
    wid = lax.axis_index("s") * NC + lax.axis_index("c")
    my_base = wid * ROWS_PER_W

    pltpu.sync_copy(x_hbm.at[pl.ds(my_base, ROWS_PER_W)], idx_v)
    pltpu.sync_copy(pos_hbm.at[pl.ds(0, SEQ)], pos_v)

    def gather_start(g, b):
        idx = idx_v.at[pl.ds(g * CH, CH)]
        pltpu.async_copy(emb_hbm.at[idx], rows_v.at[b], gsem.at[b])

    def gather_wait(g, b):
        idx = idx_v.at[pl.ds(g * CH, CH)]
        pltpu.make_async_copy(emb_hbm.at[idx], rows_v.at[b], gsem.at[b]).wait()

    def out_start(g, b):
        dst = out_hbm.at[pl.ds(my_base + g * CH, CH)]
        pltpu.async_copy(rows_v.at[b], dst, osem.at[b])

    def out_wait(g, b):
        dst = out_hbm.at[pl.ds(my_base + g * CH, CH)]
        pltpu.make_async_copy(rows_v.at[b], dst, osem.at[b]).wait()

    for b in range(NBUF - 1):
        gather_start(b, b)

    def group_body(grp, carry):
        for b in range(NBUF):
            g = grp * NBUF + b
            gather_wait(g, b)

            def add_row(r, c):
                for j in range(D // 16):
                    sl = pl.ds(j * 16, 16)
                    rows_v[b, r, sl] = rows_v[b, r, sl] + pos_v[r, sl]
                return c
            lax.fori_loop(0, CH, add_row, 0)

            out_start(g, b)

            b2 = (g + NBUF - 1) % NBUF

            @pl.when(g + NBUF - 1 < NSTEP)
            def _():
                @pl.when(g > 0)
                def _():
                    out_wait(g - 1, b2)
                gather_start(g + NBUF - 1, b2)
        return carry

    lax.fori_loop(0, NSTEP // NBUF, group_body, 0)

    for b in range(NBUF):
        g = NSTEP - NBUF + b
        out_wait(g, b)


@jax.jit
def kernel(x, emb_table, pos_table):
    x_flat = x.reshape(-1).astype(jnp.int32)

    mesh = plsc.VectorSubcoreMesh(core_axis_name="c", subcore_axis_name="s")
    run = pl.kernel(
        _emb_body,
        mesh=mesh,
        out_type=jax.ShapeDtypeStruct((B_TOTAL, D), jnp.float32),
        compiler_params=pltpu.CompilerParams(use_tc_tiling_on_sc=False),
        scratch_types=[
            pltpu.VMEM((ROWS_PER_W,), jnp.int32),
            pltpu.VMEM((SEQ, D), jnp.float32),
            pltpu.VMEM((NBUF, CH, D), jnp.float32),
            pltpu.SemaphoreType.DMA((NBUF,)),
            pltpu.SemaphoreType.DMA((NBUF,)),
        ],
    )
    out_flat = run(x_flat, emb_table, pos_table)
    return out_flat.reshape(BATCH, SEQ, D)
